# baseline (speedup 1.0000x reference)
import functools

import jax
import jax.numpy as jnp
from jax import lax
from jax.experimental import pallas as pl
from jax.experimental.pallas import tpu as pltpu
from jax.experimental.pallas import tpu_sc as plsc

N = 10000
E = 320000
F = 128
HID = 128
NCLS = 64

SC_CORES = 2
SC_SUBCORES = 16
NW = SC_CORES * SC_SUBCORES

CHUNK = 128
RING = 5
GDEPTH = 3
SDEPTH = RING - GDEPTH
NCH = 160
OUTER = NCH // RING
EP = NCH * CHUNK
E_PAD = EP * SC_SUBCORES
ROWS_PER_SUB = 640
N_PAD = ROWS_PER_SUB * SC_SUBCORES

DEG_NCH = E_PAD // (NW * CHUNK)


@functools.lru_cache(maxsize=None)
def _get_mesh():
  return plsc.VectorSubcoreMesh(
      core_axis_name="c", subcore_axis_name="s",
      num_cores=SC_CORES, num_subcores=SC_SUBCORES)


@functools.lru_cache(maxsize=None)
def _make_deg_kernel():
  D = 16

  @functools.partial(
      pl.kernel,
      out_type=jax.ShapeDtypeStruct((SC_CORES, N_PAD, D), jnp.float32),
      mesh=_get_mesh(),
      compiler_params=pltpu.CompilerParams(use_tc_tiling_on_sc=False),
      scratch_types=[
          pltpu.VMEM_SHARED((N_PAD, D), jnp.float32),
          pltpu.VMEM((CHUNK, D), jnp.float32),
          pltpu.VMEM((DEG_NCH, CHUNK), jnp.int32),
      ],
  )
  def deg_kernel(dst_hbm, e1_hbm, zeros_hbm, out_hbm, acc_sh, e1_v, dst_all):
    cid = lax.axis_index("c")
    sid = lax.axis_index("s")
    wid = sid * SC_CORES + cid
    rbase = sid * ROWS_PER_SUB
    pltpu.sync_copy(zeros_hbm, acc_sh.at[pl.ds(rbase, ROWS_PER_SUB)])
    pltpu.sync_copy(e1_hbm, e1_v)
    pltpu.sync_copy(dst_hbm.at[wid], dst_all)
    plsc.subcore_barrier()

    def body(k, carry):
      pltpu.sync_copy(e1_v, acc_sh.at[dst_all.at[k]], add=True)
      return carry

    lax.fori_loop(0, DEG_NCH, body, 0)
    plsc.subcore_barrier()
    pltpu.sync_copy(acc_sh.at[pl.ds(rbase, ROWS_PER_SUB)],
                    out_hbm.at[cid, pl.ds(rbase, ROWS_PER_SUB)])

  return deg_kernel


@functools.lru_cache(maxsize=None)
def _make_scatter_kernel(DH):
  @functools.partial(
      pl.kernel,
      out_type=jax.ShapeDtypeStruct((SC_CORES, N_PAD, DH), jnp.float32),
      mesh=_get_mesh(),
      compiler_params=pltpu.CompilerParams(use_tc_tiling_on_sc=False),
      scratch_types=[
          pltpu.VMEM_SHARED((N_PAD, DH), jnp.float32),
          pltpu.VMEM((NCH, CHUNK), jnp.int32),
          pltpu.VMEM((NCH, CHUNK), jnp.int32),
      ] + [pltpu.VMEM((CHUNK, DH), jnp.float32)] * RING
        + [pltpu.SemaphoreType.DMA] * (2 * RING),
  )
  def scat_kernel(src_hbm, dst_hbm, rows_hbm, zeros_hbm, out_hbm,
                  acc_sh, src_all, dst_all, *bufsem):
    bufs = bufsem[:RING]
    gsem = bufsem[RING:2 * RING]
    ssem = bufsem[2 * RING:]
    cid = lax.axis_index("c")
    sid = lax.axis_index("s")
    rbase = sid * ROWS_PER_SUB
    pltpu.sync_copy(zeros_hbm, acc_sh.at[pl.ds(rbase, ROWS_PER_SUB)])
    pltpu.sync_copy(src_hbm.at[cid, sid], src_all)
    pltpu.sync_copy(dst_hbm.at[sid], dst_all)
    plsc.subcore_barrier()

    for b in range(GDEPTH):
      pltpu.async_copy(rows_hbm.at[src_all.at[b]], bufs[b], gsem[b])

    def body(j, carry):
      for b in range(RING):
        k = j * RING + b
        pltpu.make_async_copy(rows_hbm.at[src_all.at[k]], bufs[b], gsem[b]).wait()
        pltpu.async_copy(bufs[b], acc_sh.at[dst_all.at[k]], ssem[b], add=True)

        bw = (b - SDEPTH) % RING

        @pl.when(k >= SDEPTH)
        def _():
          kw = k - SDEPTH
          pltpu.make_async_copy(bufs[bw], acc_sh.at[dst_all.at[kw]], ssem[bw]).wait()

        bn = (b + GDEPTH) % RING

        @pl.when(k + GDEPTH < NCH)
        def _():
          kn = k + GDEPTH
          pltpu.async_copy(rows_hbm.at[src_all.at[kn]], bufs[bn], gsem[bn])

      return carry

    lax.fori_loop(0, OUTER, body, 0)
    for k in range(NCH - SDEPTH, NCH):
      b = k % RING
      pltpu.make_async_copy(bufs[b], acc_sh.at[dst_all.at[k]], ssem[b]).wait()
    plsc.subcore_barrier()
    pltpu.sync_copy(acc_sh.at[pl.ds(rbase, ROWS_PER_SUB)],
                    out_hbm.at[cid, pl.ds(rbase, ROWS_PER_SUB)])

  return scat_kernel


@functools.lru_cache(maxsize=None)
def _make_probe_kernel():
  DH = FH
  PCH = 64
  PNCH = 160

  @functools.partial(
      pl.kernel,
      out_type=jax.ShapeDtypeStruct((SC_CORES, N_PAD, DH), jnp.float32),
      mesh=_get_mesh(),
      compiler_params=pltpu.CompilerParams(use_tc_tiling_on_sc=False),
      scratch_types=[
          pltpu.VMEM_SHARED((N_PAD, DH), jnp.float32),
          pltpu.VMEM((PNCH, PCH), jnp.int32),
      ] + [pltpu.VMEM((PCH, 128), jnp.float32)] * RING
        + [pltpu.SemaphoreType.DMA] * RING,
  )
  def probe_kernel(src_hbm, rows_hbm, zeros_hbm, out_hbm,
                   acc_sh, src_all, *bufsem):
    bufs = bufsem[:RING]
    gsem = bufsem[RING:]
    cid = lax.axis_index("c")
    sid = lax.axis_index("s")
    rbase = sid * ROWS_PER_SUB
    pltpu.sync_copy(zeros_hbm, acc_sh.at[pl.ds(rbase, ROWS_PER_SUB)])
    pltpu.sync_copy(src_hbm.at[cid, sid], src_all)
    plsc.subcore_barrier()

    for b in range(RING):
      pltpu.async_copy(rows_hbm.at[src_all.at[b]], bufs[b], gsem[b])

    def body(j, carry):
      for b in range(RING):
        k = j * RING + b
        pltpu.make_async_copy(rows_hbm.at[src_all.at[k]], bufs[b], gsem[b]).wait()

        @pl.when(k + RING < PNCH)
        def _():
          pltpu.async_copy(rows_hbm.at[src_all.at[k + RING]], bufs[b], gsem[b])

      return carry

    lax.fori_loop(0, PNCH // RING, body, 0)
    plsc.subcore_barrier()
    pltpu.sync_copy(acc_sh.at[pl.ds(rbase, ROWS_PER_SUB)],
                    out_hbm.at[cid, pl.ds(rbase, ROWS_PER_SUB)])

  return probe_kernel

TB = 1024
FH = F // 2
ZH = NCLS // 2


def _tc_prep_body(dp_ref, x_ref, y1_ref, dinv_ref):
  deg = 1.0 + dp_ref[0, :, 0:1] + dp_ref[1, :, 0:1]
  dinv = lax.rsqrt(deg)
  dinvf = jnp.broadcast_to(dinv, (TB, F))
  dinv_ref[...] = dinvf
  y1 = x_ref[...] * dinvf
  y1_ref[0] = y1[:, :FH]
  y1_ref[1] = y1[:, FH:]


def _tc_mid_body(s1_ref, y1_ref, dinv_ref, w1_ref, b1_ref, w2_ref, z_ref):
  dinvf = dinv_ref[...]
  s1 = jnp.concatenate([s1_ref[0] + y1_ref[0], s1_ref[1] + y1_ref[1]], axis=1)
  agg1 = dinvf * s1
  h = lax.dot_general(agg1, w1_ref[...], (((1,), (0,)), ((), ())),
                      preferred_element_type=jnp.float32)
  h = jnp.maximum(h + b1_ref[...], 0.0)
  z = lax.dot_general(h * dinvf, w2_ref[...], (((1,), (0,)), ((), ())),
                      preferred_element_type=jnp.float32)
  z_ref[0] = z[:, :ZH]
  z_ref[1] = z[:, ZH:]


def _tc_out_body(s2_ref, z_ref, dinv_ref, b2_ref, out_ref):
  s2 = jnp.concatenate([s2_ref[0] + z_ref[0], s2_ref[1] + z_ref[1]], axis=1)
  out_ref[...] = dinv_ref[...][:, :NCLS] * s2 + b2_ref[...]


def _tc_prep(dp, x_pad):
  grid = (N_PAD // TB,)
  return pl.pallas_call(
      _tc_prep_body,
      grid=grid,
      in_specs=[
          pl.BlockSpec((SC_CORES, TB, 16), lambda i: (0, i, 0)),
          pl.BlockSpec((TB, F), lambda i: (i, 0)),
      ],
      out_specs=[
          pl.BlockSpec((SC_CORES, TB, FH), lambda i: (0, i, 0)),
          pl.BlockSpec((TB, F), lambda i: (i, 0)),
      ],
      out_shape=[
          jax.ShapeDtypeStruct((SC_CORES, N_PAD, FH), jnp.float32),
          jax.ShapeDtypeStruct((N_PAD, F), jnp.float32),
      ],
  )(dp, x_pad)


def _tc_mid(s1, y1c, dinvf, W1, b1, W2):
  grid = (N_PAD // TB,)
  return pl.pallas_call(
      _tc_mid_body,
      grid=grid,
      in_specs=[
          pl.BlockSpec((SC_CORES, TB, FH), lambda i: (0, i, 0)),
          pl.BlockSpec((SC_CORES, TB, FH), lambda i: (0, i, 0)),
          pl.BlockSpec((TB, F), lambda i: (i, 0)),
          pl.BlockSpec((F, HID), lambda i: (0, 0)),
          pl.BlockSpec((1, HID), lambda i: (0, 0)),
          pl.BlockSpec((HID, NCLS), lambda i: (0, 0)),
      ],
      out_specs=pl.BlockSpec((SC_CORES, TB, ZH), lambda i: (0, i, 0)),
      out_shape=jax.ShapeDtypeStruct((SC_CORES, N_PAD, ZH), jnp.float32),
  )(s1, y1c, dinvf, W1, b1.reshape(1, HID), W2)


def _tc_out(s2, zc, dinvf, b2):
  grid = (N_PAD // TB,)
  return pl.pallas_call(
      _tc_out_body,
      grid=grid,
      in_specs=[
          pl.BlockSpec((SC_CORES, TB, ZH), lambda i: (0, i, 0)),
          pl.BlockSpec((SC_CORES, TB, ZH), lambda i: (0, i, 0)),
          pl.BlockSpec((TB, F), lambda i: (i, 0)),
          pl.BlockSpec((1, NCLS), lambda i: (0, 0)),
      ],
      out_specs=pl.BlockSpec((TB, NCLS), lambda i: (i, 0)),
      out_shape=jax.ShapeDtypeStruct((N_PAD, NCLS), jnp.float32),
  )(s2, zc, dinvf, b2.reshape(1, NCLS))


@jax.jit
def kernel(node_features, edge_indices, W1, b1, W2, b2):
  pad = E_PAD - E
  srcp = jnp.concatenate(
      [edge_indices[0], jnp.full((pad,), N, jnp.int32)])
  dstp = jnp.concatenate(
      [edge_indices[1], jnp.full((pad,), N, jnp.int32)])
  src_sub = srcp.reshape(SC_SUBCORES, NCH, CHUNK)
  src2 = jnp.stack([src_sub, src_sub + N_PAD])
  dst_sub = dstp.reshape(SC_SUBCORES, NCH, CHUNK)
  dst_deg = dstp.reshape(NW, DEG_NCH, CHUNK)

  x_pad = jnp.pad(node_features, ((0, N_PAD - N), (0, 0)))

  e1 = jnp.zeros((CHUNK, 16), jnp.float32).at[:, 0].set(1.0)
  z16 = jnp.zeros((ROWS_PER_SUB, 16), jnp.float32)
  zfh = jnp.zeros((ROWS_PER_SUB, FH), jnp.float32)
  zzh = jnp.zeros((ROWS_PER_SUB, ZH), jnp.float32)

  dp = _make_deg_kernel()(dst_deg, e1, z16)
  y1c, dinvf = _tc_prep(dp, x_pad)
  src_probe = srcp.reshape(2, SC_SUBCORES, 160, 64)
  s1 = _make_probe_kernel()(src_probe, dinvf, zfh)
  zc = _tc_mid(s1, y1c, dinvf, W1, b1, W2)
  s2 = _make_scatter_kernel(ZH)(
      src2, dst_sub, zc.reshape(2 * N_PAD, ZH), zzh)
  outp = _tc_out(s2, zc, dinvf, b2)
  return outp[:N]


# speedup vs baseline: 2.5569x; 2.5569x over previous
import functools

import jax
import jax.numpy as jnp
from jax import lax
from jax.experimental import pallas as pl
from jax.experimental.pallas import tpu as pltpu
from jax.experimental.pallas import tpu_sc as plsc

N = 10000
E = 320000
F = 128
HID = 128
NCLS = 64

SC_CORES = 2
SC_SUBCORES = 16
NW = SC_CORES * SC_SUBCORES

CHUNK = 64
NCH = 160
EP = NCH * CHUNK
E_PAD = EP * NW
ROWS_PER_SUB = 640
N_PAD = ROWS_PER_SUB * SC_SUBCORES

DEG_NCH = E_PAD // (NW * 128)


@functools.lru_cache(maxsize=None)
def _get_mesh():
  return plsc.VectorSubcoreMesh(
      core_axis_name="c", subcore_axis_name="s",
      num_cores=SC_CORES, num_subcores=SC_SUBCORES)


@functools.lru_cache(maxsize=None)
def _make_deg_kernel():
  D = 16

  @functools.partial(
      pl.kernel,
      out_type=jax.ShapeDtypeStruct((SC_CORES, N_PAD, D), jnp.float32),
      mesh=_get_mesh(),
      compiler_params=pltpu.CompilerParams(use_tc_tiling_on_sc=False),
      scratch_types=[
          pltpu.VMEM_SHARED((N_PAD, D), jnp.float32),
          pltpu.VMEM((128, D), jnp.float32),
          pltpu.VMEM((DEG_NCH, 128), jnp.int32),
      ],
  )
  def deg_kernel(dst_hbm, e1_hbm, zeros_hbm, out_hbm, acc_sh, e1_v, dst_all):
    cid = lax.axis_index("c")
    sid = lax.axis_index("s")
    wid = sid * SC_CORES + cid
    rbase = sid * ROWS_PER_SUB
    pltpu.sync_copy(zeros_hbm, acc_sh.at[pl.ds(rbase, ROWS_PER_SUB)])
    pltpu.sync_copy(e1_hbm, e1_v)
    pltpu.sync_copy(dst_hbm.at[wid], dst_all)
    plsc.subcore_barrier()

    def body(k, carry):
      pltpu.sync_copy(e1_v, acc_sh.at[dst_all.at[k]], add=True)
      return carry

    lax.fori_loop(0, DEG_NCH, body, 0)
    plsc.subcore_barrier()
    pltpu.sync_copy(acc_sh.at[pl.ds(rbase, ROWS_PER_SUB)],
                    out_hbm.at[cid, pl.ds(rbase, ROWS_PER_SUB)])

  return deg_kernel


@functools.lru_cache(maxsize=None)
def _make_scatter_kernel(D, ring, gdepth):
  sdepth = ring - gdepth

  @functools.partial(
      pl.kernel,
      out_type=jax.ShapeDtypeStruct((SC_CORES, N_PAD, D), jnp.float32),
      mesh=_get_mesh(),
      compiler_params=pltpu.CompilerParams(use_tc_tiling_on_sc=False),
      scratch_types=[
          pltpu.VMEM_SHARED((N_PAD, D), jnp.float32),
          pltpu.VMEM((NCH, CHUNK), jnp.int32),
          pltpu.VMEM((NCH, CHUNK), jnp.int32),
      ] + [pltpu.VMEM((CHUNK, D), jnp.float32)] * ring
        + [pltpu.SemaphoreType.DMA] * (2 * ring),
  )
  def scat_kernel(src_hbm, dst_hbm, rows_hbm, zeros_hbm, out_hbm,
                  acc_sh, src_all, dst_all, *bufsem):
    bufs = bufsem[:ring]
    gsem = bufsem[ring:2 * ring]
    ssem = bufsem[2 * ring:]
    cid = lax.axis_index("c")
    sid = lax.axis_index("s")
    rbase = sid * ROWS_PER_SUB
    pltpu.sync_copy(zeros_hbm, acc_sh.at[pl.ds(rbase, ROWS_PER_SUB)])
    pltpu.sync_copy(src_hbm.at[cid, sid], src_all)
    pltpu.sync_copy(dst_hbm.at[cid, sid], dst_all)
    plsc.subcore_barrier()

    for b in range(gdepth):
      pltpu.async_copy(rows_hbm.at[src_all.at[b]], bufs[b], gsem[b])

    def body(j, carry):
      for b in range(ring):
        k = j * ring + b
        pltpu.make_async_copy(rows_hbm.at[src_all.at[k]], bufs[b], gsem[b]).wait()
        pltpu.async_copy(bufs[b], acc_sh.at[dst_all.at[k]], ssem[b], add=True)

        bw = (b - sdepth) % ring

        @pl.when(k >= sdepth)
        def _():
          kw = k - sdepth
          pltpu.make_async_copy(bufs[bw], acc_sh.at[dst_all.at[kw]], ssem[bw]).wait()

        bn = (b + gdepth) % ring

        @pl.when(k + gdepth < NCH)
        def _():
          kn = k + gdepth
          pltpu.async_copy(rows_hbm.at[src_all.at[kn]], bufs[bn], gsem[bn])

      return carry

    lax.fori_loop(0, NCH // ring, body, 0)
    tail = NCH % ring
    for k in range(NCH - tail, NCH):
      b = k % ring
      pltpu.make_async_copy(rows_hbm.at[src_all.at[k]], bufs[b], gsem[b]).wait()
      pltpu.async_copy(bufs[b], acc_sh.at[dst_all.at[k]], ssem[b], add=True)
    for k in range(NCH - tail - sdepth, NCH):
      b = k % ring
      pltpu.make_async_copy(bufs[b], acc_sh.at[dst_all.at[k]], ssem[b]).wait()
    plsc.subcore_barrier()
    pltpu.sync_copy(acc_sh.at[pl.ds(rbase, ROWS_PER_SUB)],
                    out_hbm.at[cid, pl.ds(rbase, ROWS_PER_SUB)])

  return scat_kernel


TB = 1024


def _tc_prep_body(dp_ref, x_ref, y1_ref, dinv_ref):
  deg = 1.0 + dp_ref[0, :, 0:1] + dp_ref[1, :, 0:1]
  dinv = lax.rsqrt(deg)
  dinvf = jnp.broadcast_to(dinv, (TB, F))
  dinv_ref[...] = dinvf
  y1_ref[...] = x_ref[...] * dinvf


def _tc_mid_body(s1_ref, y1_ref, dinv_ref, w1_ref, b1_ref, w2_ref, z_ref):
  dinvf = dinv_ref[...]
  s1 = s1_ref[0] + s1_ref[1] + y1_ref[...]
  agg1 = dinvf * s1
  h = lax.dot_general(agg1, w1_ref[...], (((1,), (0,)), ((), ())),
                      preferred_element_type=jnp.float32)
  h = jnp.maximum(h + b1_ref[...], 0.0)
  z_ref[...] = lax.dot_general(h * dinvf, w2_ref[...], (((1,), (0,)), ((), ())),
                               preferred_element_type=jnp.float32)


def _tc_out_body(s2_ref, z_ref, dinv_ref, b2_ref, out_ref):
  s2 = s2_ref[0] + s2_ref[1] + z_ref[...]
  out_ref[...] = dinv_ref[...][:, :NCLS] * s2 + b2_ref[...]


def _tc_prep(dp, x_pad):
  grid = (N_PAD // TB,)
  return pl.pallas_call(
      _tc_prep_body,
      grid=grid,
      in_specs=[
          pl.BlockSpec((SC_CORES, TB, 16), lambda i: (0, i, 0)),
          pl.BlockSpec((TB, F), lambda i: (i, 0)),
      ],
      out_specs=[
          pl.BlockSpec((TB, F), lambda i: (i, 0)),
          pl.BlockSpec((TB, F), lambda i: (i, 0)),
      ],
      out_shape=[
          jax.ShapeDtypeStruct((N_PAD, F), jnp.float32),
          jax.ShapeDtypeStruct((N_PAD, F), jnp.float32),
      ],
  )(dp, x_pad)


def _tc_mid(s1, y1p, dinvf, W1, b1, W2):
  grid = (N_PAD // TB,)
  return pl.pallas_call(
      _tc_mid_body,
      grid=grid,
      in_specs=[
          pl.BlockSpec((SC_CORES, TB, F), lambda i: (0, i, 0)),
          pl.BlockSpec((TB, F), lambda i: (i, 0)),
          pl.BlockSpec((TB, F), lambda i: (i, 0)),
          pl.BlockSpec((F, HID), lambda i: (0, 0)),
          pl.BlockSpec((1, HID), lambda i: (0, 0)),
          pl.BlockSpec((HID, NCLS), lambda i: (0, 0)),
      ],
      out_specs=pl.BlockSpec((TB, NCLS), lambda i: (i, 0)),
      out_shape=jax.ShapeDtypeStruct((N_PAD, NCLS), jnp.float32),
  )(s1, y1p, dinvf, W1, b1.reshape(1, HID), W2)


def _tc_out(s2, zp, dinvf, b2):
  grid = (N_PAD // TB,)
  return pl.pallas_call(
      _tc_out_body,
      grid=grid,
      in_specs=[
          pl.BlockSpec((SC_CORES, TB, NCLS), lambda i: (0, i, 0)),
          pl.BlockSpec((TB, NCLS), lambda i: (i, 0)),
          pl.BlockSpec((TB, F), lambda i: (i, 0)),
          pl.BlockSpec((1, NCLS), lambda i: (0, 0)),
      ],
      out_specs=pl.BlockSpec((TB, NCLS), lambda i: (i, 0)),
      out_shape=jax.ShapeDtypeStruct((N_PAD, NCLS), jnp.float32),
  )(s2, zp, dinvf, b2.reshape(1, NCLS))


@jax.jit
def kernel(node_features, edge_indices, W1, b1, W2, b2):
  pad = E_PAD - E
  fill = jnp.arange(pad, dtype=jnp.int32)
  srcp = jnp.concatenate([edge_indices[0], fill % N])
  dstp = jnp.concatenate([edge_indices[1], N + fill % (N_PAD - N)])
  src4 = srcp.reshape(SC_CORES, SC_SUBCORES, NCH, CHUNK)
  dst4 = dstp.reshape(SC_CORES, SC_SUBCORES, NCH, CHUNK)
  dst_deg = dstp.reshape(NW, DEG_NCH, 128)

  x_pad = jnp.pad(node_features, ((0, N_PAD - N), (0, 0)))

  e1 = jnp.zeros((128, 16), jnp.float32).at[:, 0].set(1.0)
  z16 = jnp.zeros((ROWS_PER_SUB, 16), jnp.float32)
  z64 = jnp.zeros((ROWS_PER_SUB, 64), jnp.float32)
  z128 = jnp.zeros((ROWS_PER_SUB, 128), jnp.float32)

  dp = _make_deg_kernel()(dst_deg, e1, z16)
  y1p, dinvf = _tc_prep(dp, x_pad)
  s1 = _make_scatter_kernel(128, 3, 2)(src4, dst4, y1p, z128)
  zp = _tc_mid(s1, y1p, dinvf, W1, b1, W2)
  s2 = _make_scatter_kernel(64, 4, 3)(src4, dst4, zp, z64)
  outp = _tc_out(s2, zp, dinvf, b2)
  return outp[:N]
